# Initial kernel scaffold; baseline (speedup 1.0000x reference)
import jax
import jax.numpy as jnp
from jax.experimental import pallas as pl


def kernel(x, table):
    raise NotImplementedError("write your pallas kernel here")


# speedup vs baseline: 3.4856x; 3.4856x over previous
import jax
import jax.numpy as jnp
from jax import lax
from jax.experimental import pallas as pl
from jax.experimental.pallas import tpu as pltpu
from jax.experimental.pallas import tpu_sc as plsc

B = 16384
L = 200
E = 16
V = 1000000

NC, NS = 2, 16
NW = NC * NS
R = 4
IDX_PER_CHUNK = R * L
IDX_MINOR = 100
IDX_ROWS = IDX_PER_CHUNK // IDX_MINOR
OUT_PER_CHUNK = R * E * L
CHUNKS_PER_WORKER = (B // NW) // R

_mesh = plsc.VectorSubcoreMesh(
    core_axis_name="c", subcore_axis_name="s", num_cores=NC, num_subcores=NS
)


def _body(x_hbm, table_hbm, out_hbm, idx_v, rows_v, outT, sem):
    wid = lax.axis_index("s") * NC + lax.axis_index("c")
    eoff = lax.iota(jnp.int32, 16) * L

    def chunk(c, carry):
        g = wid * CHUNKS_PER_WORKER + c
        row0 = g * IDX_ROWS
        pltpu.sync_copy(x_hbm.at[pl.ds(row0, IDX_ROWS)], idx_v)
        cps = [
            pltpu.async_copy(
                table_hbm.at[idx_v.at[j]],
                rows_v.at[pl.ds(j * IDX_MINOR, IDX_MINOR)],
                sem,
            )
            for j in range(IDX_ROWS)
        ]
        for cp in cps:
            cp.wait()

        def scat(i, carry2):
            for j in range(IDX_ROWS):
                vec = rows_v[j * IDX_MINOR + i, :]
                base = (j // 2) * (E * L) + (j % 2) * IDX_MINOR
                plsc.store_scatter(outT, [eoff + (base + i)], vec)
            return carry2

        lax.fori_loop(0, IDX_MINOR, scat, 0)
        pltpu.sync_copy(outT, out_hbm.at[pl.ds(g * OUT_PER_CHUNK, OUT_PER_CHUNK)])
        return carry

    lax.fori_loop(0, CHUNKS_PER_WORKER, chunk, 0)


_gather_transpose = pl.kernel(
    _body,
    out_type=jax.ShapeDtypeStruct((B * E * L,), jnp.float32),
    mesh=_mesh,
    scratch_types=[
        pltpu.VMEM((IDX_ROWS, IDX_MINOR), jnp.int32),
        pltpu.VMEM((IDX_PER_CHUNK, E), jnp.float32),
        pltpu.VMEM((OUT_PER_CHUNK,), jnp.float32),
        pltpu.SemaphoreType.DMA,
    ],
    compiler_params=pltpu.CompilerParams(
        needs_layout_passes=False, use_tc_tiling_on_sc=False
    ),
)


@jax.jit
def kernel(x, table):
    x2d = x.reshape(B * L // IDX_MINOR, IDX_MINOR)
    out_flat = _gather_transpose(x2d, table)
    return out_flat.reshape(B, E, L)


# speedup vs baseline: 4.7014x; 1.3488x over previous
import jax
import jax.numpy as jnp
from jax import lax
from jax.experimental import pallas as pl
from jax.experimental.pallas import tpu as pltpu
from jax.experimental.pallas import tpu_sc as plsc

B = 16384
L = 200
E = 16
V = 1000000

NC, NS = 2, 16
NW = NC * NS
R = 8
IDX_PER_CHUNK = R * L
IDX_MINOR = 100
IDX_ROWS = IDX_PER_CHUNK // IDX_MINOR
OUT_PER_CHUNK = R * E * L
CHUNKS_PER_WORKER = (B // NW) // R

_mesh = plsc.VectorSubcoreMesh(
    core_axis_name="c", subcore_axis_name="s", num_cores=NC, num_subcores=NS
)


def _body(x_hbm, table_hbm, out_hbm, idx0, idx1, rows0, rows1, outT0, outT1,
          sem0, sem1, osem0, osem1):
    wid = lax.axis_index("s") * NC + lax.axis_index("c")
    eoff = lax.iota(jnp.int32, 16) * L
    c_base = wid * CHUNKS_PER_WORKER

    def fire(idx_v, rows_v, sem, c):
        pltpu.sync_copy(x_hbm.at[pl.ds((c_base + c) * IDX_ROWS, IDX_ROWS)], idx_v)
        for j in range(IDX_ROWS):
            pltpu.async_copy(
                table_hbm.at[idx_v.at[j]],
                rows_v.at[pl.ds(j * IDX_MINOR, IDX_MINOR)],
                sem,
            )

    def drain_gathers(rows_v, sem):
        pltpu.make_async_copy(table_hbm.at[pl.ds(0, IDX_PER_CHUNK)], rows_v, sem).wait()

    def out_slice(c):
        return out_hbm.at[pl.ds((c_base + c) * OUT_PER_CHUNK, OUT_PER_CHUNK)]

    def drain_out(outT, osem):
        pltpu.make_async_copy(outT, out_slice(0), osem).wait()

    def scat(rows_v, outT):
        @plsc.parallel_loop(0, IDX_MINOR, unroll=2)
        def _(i):
            for j in range(IDX_ROWS):
                vec = rows_v[j * IDX_MINOR + i, :]
                base = (j // 2) * (E * L) + (j % 2) * IDX_MINOR
                plsc.store_scatter(outT, [eoff + (base + i)], vec)

    fire(idx0, rows0, sem0, 0)

    def step(t, carry):
        c0 = 2 * t
        c1 = 2 * t + 1
        drain_gathers(rows0, sem0)
        fire(idx1, rows1, sem1, c1)

        @pl.when(t > 0)
        def _():
            drain_out(outT0, osem0)

        scat(rows0, outT0)
        pltpu.async_copy(outT0, out_slice(c0), osem0)

        drain_gathers(rows1, sem1)

        @pl.when(t < CHUNKS_PER_WORKER // 2 - 1)
        def _():
            fire(idx0, rows0, sem0, c0 + 2)

        @pl.when(t > 0)
        def _():
            drain_out(outT1, osem1)

        scat(rows1, outT1)
        pltpu.async_copy(outT1, out_slice(c1), osem1)
        return carry

    lax.fori_loop(0, CHUNKS_PER_WORKER // 2, step, 0)
    drain_out(outT0, osem0)
    drain_out(outT1, osem1)


_gather_transpose = pl.kernel(
    _body,
    out_type=jax.ShapeDtypeStruct((B * E * L,), jnp.float32),
    mesh=_mesh,
    scratch_types=[
        pltpu.VMEM((IDX_ROWS, IDX_MINOR), jnp.int32),
        pltpu.VMEM((IDX_ROWS, IDX_MINOR), jnp.int32),
        pltpu.VMEM((IDX_PER_CHUNK, E), jnp.float32),
        pltpu.VMEM((IDX_PER_CHUNK, E), jnp.float32),
        pltpu.VMEM((OUT_PER_CHUNK,), jnp.float32),
        pltpu.VMEM((OUT_PER_CHUNK,), jnp.float32),
        pltpu.SemaphoreType.DMA,
        pltpu.SemaphoreType.DMA,
        pltpu.SemaphoreType.DMA,
        pltpu.SemaphoreType.DMA,
    ],
    compiler_params=pltpu.CompilerParams(
        needs_layout_passes=False, use_tc_tiling_on_sc=False
    ),
)


@jax.jit
def kernel(x, table):
    x2d = x.reshape(B * L // IDX_MINOR, IDX_MINOR)
    out_flat = _gather_transpose(x2d, table)
    return out_flat.reshape(B, E, L)


# speedup vs baseline: 9.7113x; 2.0656x over previous
import jax
import jax.numpy as jnp
from jax import lax
from jax.experimental import pallas as pl
from jax.experimental.pallas import tpu as pltpu
from jax.experimental.pallas import tpu_sc as plsc

B = 16384
L = 200
E = 16
V = 1000000

NC, NS = 2, 16
NW = NC * NS
TI = L // 8
TJ = B // 128
NT = TI * TJ
TILE = 1024
TPAD = TILE + 1
IDX_ROWS = 8
TPW = NT // NW

_mesh = plsc.VectorSubcoreMesh(
    core_axis_name="c", subcore_axis_name="s", num_cores=NC, num_subcores=NS
)


def _body(x_hbm, table_hbm, out_hbm,
          idx_s, rows_s, outT_s, isem_s, gsem_s, osem_s):
    wid = lax.axis_index("s") * NC + lax.axis_index("c")
    eoff = lax.iota(jnp.int32, 16)
    t_base = wid * TPW

    def fire_idx(b, t):
        pltpu.async_copy(
            x_hbm.at[pl.ds((t_base + t) * IDX_ROWS, IDX_ROWS)],
            idx_s[b], isem_s[b],
        )

    def fire_gat(b, t):
        del t
        pltpu.make_async_copy(x_hbm.at[pl.ds(0, IDX_ROWS)], idx_s[b],
                              isem_s[b]).wait()
        for j in range(IDX_ROWS):
            pltpu.async_copy(
                table_hbm.at[idx_s[b].at[j]],
                rows_s[b].at[pl.ds(j * 128, 128)],
                gsem_s[b],
            )

    def drain_gathers(b):
        pltpu.make_async_copy(table_hbm.at[pl.ds(0, TILE)], rows_s[b],
                              gsem_s[b]).wait()

    def out_slice(t):
        return out_hbm.at[:, pl.ds((t_base + t) * TILE, TILE)]

    def drain_out(ob):
        pltpu.make_async_copy(outT_s[ob].at[:, pl.ds(0, TILE)], out_slice(0),
                              osem_s[ob]).wait()

    def scat(b, ob):
        rows_v = rows_s[b]
        outT = outT_s[ob]

        @plsc.parallel_loop(0, TILE // 16, unroll=2)
        def _(k16):
            base = k16 * 16
            for kk in range(16):
                k = base + kk
                vec = rows_v[k, :]
                plsc.store_scatter(outT, [eoff, jnp.broadcast_to(k, (16,))], vec)

    for b in range(4):
        fire_idx(b, b)
    fire_gat(0, 0)
    fire_gat(1, 1)

    def step(q, carry):
        for b in range(4):
            t = 4 * q + b
            ob = b % 2
            drain_gathers(b)

            @pl.when(t + 2 < TPW)
            def _():
                fire_gat((b + 2) % 4, t + 2)

            @pl.when(t > 1)
            def _():
                drain_out(ob)

            scat(b, ob)
            pltpu.async_copy(outT_s[ob].at[:, pl.ds(0, TILE)], out_slice(t),
                             osem_s[ob])

            @pl.when(t + 4 < TPW)
            def _():
                fire_idx(b, t + 4)
        return carry

    lax.fori_loop(0, TPW // 4, step, 0)
    drain_out(0)
    drain_out(1)


_gather_transpose = pl.kernel(
    _body,
    out_type=jax.ShapeDtypeStruct((E, NT * TILE), jnp.float32),
    mesh=_mesh,
    scratch_types=[
        [pltpu.VMEM((IDX_ROWS, 128), jnp.int32) for _ in range(4)],
        [pltpu.VMEM((TILE, E), jnp.float32) for _ in range(4)],
        [pltpu.VMEM((E, TPAD), jnp.float32) for _ in range(2)],
        [pltpu.SemaphoreType.DMA for _ in range(4)],
        [pltpu.SemaphoreType.DMA for _ in range(4)],
        [pltpu.SemaphoreType.DMA for _ in range(2)],
    ],
    compiler_params=pltpu.CompilerParams(
        needs_layout_passes=False, use_tc_tiling_on_sc=False
    ),
)


@jax.jit
def kernel(x, table):
    x_pre = (
        x.transpose(1, 0)
        .reshape(TI, 8, TJ, 128)
        .transpose(0, 2, 1, 3)
        .reshape(NT * IDX_ROWS, 128)
    )
    out2d = _gather_transpose(x_pre, table)
    out = (
        out2d.reshape(E, TI, TJ, 8, 128)
        .transpose(2, 4, 0, 1, 3)
        .reshape(B, E, L)
    )
    return out
